# Initial kernel scaffold; baseline (speedup 1.0000x reference)
import jax
import jax.numpy as jnp
from jax.experimental import pallas as pl


def kernel(X, word_table, pos_table):
    raise NotImplementedError("write your pallas kernel here")


# speedup vs baseline: 5.5260x; 5.5260x over previous
import functools

import jax
import jax.numpy as jnp
from jax import lax
from jax.experimental import pallas as pl
from jax.experimental.pallas import tpu as pltpu
from jax.experimental.pallas import tpu_sc as plsc

B, T, V, D = 16384, 12, 28, 24
N = B * T
LANES = 16

NUM_CORES = 2
NUM_SUBCORES = 16
NW = NUM_CORES * NUM_SUBCORES
PER_W = N // NW
HALF = PER_W // 2
GCH = 128
N_G = HALF // GCH
N_TV = HALF // LANES


def _fuse_body(word_ref, pos_ref, out_ref):
    out_ref[...] = pos_ref[...][:, None, :] + word_ref[...][None, :, :]


def _build_fused(word_table, pos_table):
    return pl.pallas_call(
        _fuse_body,
        out_shape=jax.ShapeDtypeStruct((T, V, D), jnp.float32),
    )(word_table, pos_table)


def _sc_body(x_hbm, fused_hbm, out_hbm, idx_v, rows_v, sem):
    wid = lax.axis_index("s") * NUM_CORES + lax.axis_index("c")
    lane = lax.iota(jnp.int32, LANES)

    for h in range(2):
        base = wid * PER_W + h * HALF

        pltpu.sync_copy(x_hbm.at[pl.ds(base, HALF)], idx_v)

        def transform(i, _):
            off = i * LANES
            p = off + lane
            t = lax.rem(p, T)
            idx_v[pl.ds(off, LANES)] = idx_v[pl.ds(off, LANES)] + t * V
            return 0

        lax.fori_loop(0, N_TV, transform, 0)

        copies = []
        for j in range(N_G):
            copies.append(
                pltpu.async_copy(
                    fused_hbm.at[idx_v.at[pl.ds(j * GCH, GCH)]],
                    rows_v.at[pl.ds(j * GCH, GCH), :],
                    sem,
                )
            )
        for c in copies:
            c.wait()

        pltpu.sync_copy(rows_v, out_hbm.at[pl.ds(base, HALF), :])


@jax.jit
def kernel(X, word_table, pos_table):
    fused = _build_fused(word_table, pos_table).reshape(T * V, D)
    x_flat = X.reshape(N)

    mesh = plsc.VectorSubcoreMesh(core_axis_name="c", subcore_axis_name="s")
    sc = pl.kernel(
        _sc_body,
        out_type=jax.ShapeDtypeStruct((N, D), jnp.float32),
        mesh=mesh,
        scratch_types=[
            pltpu.VMEM((HALF,), jnp.int32),
            pltpu.VMEM((HALF, D), jnp.float32),
            pltpu.SemaphoreType.DMA,
        ],
        compiler_params=pltpu.CompilerParams(use_tc_tiling_on_sc=False),
    )
    out = sc(x_flat, fused)
    return out.reshape(B, T, D)


# speedup vs baseline: 10.7432x; 1.9441x over previous
import functools

import jax
import jax.numpy as jnp
from jax import lax
from jax.experimental import pallas as pl
from jax.experimental.pallas import tpu as pltpu
from jax.experimental.pallas import tpu_sc as plsc

B, T, V, D = 16384, 12, 28, 24
LANES = 16

NUM_CORES = 2
NUM_SUBCORES = 16
NW = NUM_CORES * NUM_SUBCORES
BPW = B // NW
BT = 128
UNITS = BPW // BT
DT = D // 8


def _sc_body(x_hbm, word_hbm, pos_hbm, out_hbm, xch_v, wt_v, pos_v, ft_v, blk_v, sem):
    wid = lax.axis_index("s") * NUM_CORES + lax.axis_index("c")
    b0 = wid * BPW

    pltpu.sync_copy(word_hbm, wt_v)
    pltpu.sync_copy(pos_hbm, pos_v)
    pltpu.sync_copy(x_hbm.at[pl.ds(b0 * T, BPW * T)], xch_v)

    lane = lax.iota(jnp.int32, LANES)
    lane12 = lane * T

    def ft_t(t, _):
        pv = [
            plsc.load_gather(pos_v, [lax.rem(k * LANES + lane, D) + t * D])
            for k in range(3)
        ]

        def ft_grp(grp, _):
            for k in range(3):
                off = (grp * 3 + k) * LANES
                ft_v[pl.ds(t * (V * D) + off, LANES)] = wt_v[pl.ds(off, LANES)] + pv[k]
            return 0

        lax.fori_loop(0, (V * D) // (3 * LANES), ft_grp, 0)
        return 0

    lax.fori_loop(0, T, ft_t, 0)

    for u in range(UNITS):
        bh = wid * UNITS + u
        xbase = u * BT * T

        def t_loop(t, _):
            def g_loop(g, _):
                xv = plsc.load_gather(xch_v, [lane12 + (xbase + g * (LANES * T) + t)])
                xvt = xv * D + t * (V * D)
                for d in range(D):
                    val = plsc.load_gather(ft_v, [xvt + d])
                    blk_v[t, d // 8, d % 8, pl.ds(g * LANES, LANES)] = val
                return 0

            lax.fori_loop(0, BT // LANES, g_loop, 0)
            return 0

        lax.fori_loop(0, T, t_loop, 0)
        pltpu.sync_copy(blk_v, out_hbm.at[:, :, bh])


@jax.jit
def kernel(X, word_table, pos_table):
    x_flat = X.reshape(B * T)
    wt_flat = word_table.reshape(V * D)
    pos_flat = pos_table.reshape(T * D)

    mesh = plsc.VectorSubcoreMesh(core_axis_name="c", subcore_axis_name="s")
    sc = pl.kernel(
        _sc_body,
        out_type=jax.ShapeDtypeStruct((T, DT, B // BT, 8, BT), jnp.float32),
        mesh=mesh,
        scratch_types=[
            pltpu.VMEM((BPW * T,), jnp.int32),
            pltpu.VMEM((V * D,), jnp.float32),
            pltpu.VMEM((T * D,), jnp.float32),
            pltpu.VMEM((T * V * D,), jnp.float32),
            pltpu.VMEM((T, DT, 8, BT), jnp.float32),
            pltpu.SemaphoreType.DMA,
        ],
        compiler_params=pltpu.CompilerParams(
            use_tc_tiling_on_sc=False, needs_layout_passes=False
        ),
    )
    out5 = sc(x_flat, wt_flat, pos_flat)
    return jnp.transpose(out5, (2, 4, 0, 1, 3)).reshape(B, T, D)


# speedup vs baseline: 10.9715x; 1.0212x over previous
import functools

import jax
import jax.numpy as jnp
from jax import lax
from jax.experimental import pallas as pl
from jax.experimental.pallas import tpu as pltpu
from jax.experimental.pallas import tpu_sc as plsc

B, T, V, D = 16384, 12, 28, 24
LANES = 16

NUM_CORES = 2
NUM_SUBCORES = 16
NW = NUM_CORES * NUM_SUBCORES
BPW = B // NW
BT = 128
UNITS = BPW // BT
DT = D // 8


def _sc_body(x_hbm, word_hbm, pos_hbm, out_hbm, xch_v, wt_v, pos_v, ft_v, blk_v, sem):
    wid = lax.axis_index("s") * NUM_CORES + lax.axis_index("c")
    b0 = wid * BPW

    pltpu.sync_copy(word_hbm, wt_v)
    pltpu.sync_copy(pos_hbm, pos_v)
    pltpu.sync_copy(x_hbm.at[pl.ds(b0 * T, BPW * T)], xch_v)

    lane = lax.iota(jnp.int32, LANES)
    lane12 = lane * T

    def ft_t(t, _):
        pv = [
            plsc.load_gather(pos_v, [lax.rem(k * LANES + lane, D) + t * D])
            for k in range(3)
        ]

        def ft_grp(grp, _):
            for k in range(3):
                off = (grp * 3 + k) * LANES
                ft_v[pl.ds(t * (V * D) + off, LANES)] = wt_v[pl.ds(off, LANES)] + pv[k]
            return 0

        lax.fori_loop(0, (V * D) // (3 * LANES), ft_grp, 0)
        return 0

    lax.fori_loop(0, T, ft_t, 0)

    copies = []
    for u in range(UNITS):
        bh = wid * UNITS + u
        xbase = u * BT * T
        buf = u % 2
        if u >= 2:
            copies[u - 2].wait()

        def t_loop(t, _):
            for g in range(BT // LANES):
                xv = plsc.load_gather(xch_v, [lane12 + (xbase + g * (LANES * T) + t)])
                xvt = xv * D + t * (V * D)
                for d in range(D):
                    val = plsc.load_gather(ft_v, [xvt + d])
                    blk_v[buf, t, d // 8, d % 8, pl.ds(g * LANES, LANES)] = val
            return 0

        lax.fori_loop(0, T, t_loop, 0)
        copies.append(pltpu.async_copy(blk_v.at[buf], out_hbm.at[:, :, bh], sem))
    for c in copies[-2:]:
        c.wait()


@jax.jit
def kernel(X, word_table, pos_table):
    x_flat = X.reshape(B * T)
    wt_flat = word_table.reshape(V * D)
    pos_flat = pos_table.reshape(T * D)

    mesh = plsc.VectorSubcoreMesh(core_axis_name="c", subcore_axis_name="s")
    sc = pl.kernel(
        _sc_body,
        out_type=jax.ShapeDtypeStruct((T, DT, B // BT, 8, BT), jnp.float32),
        mesh=mesh,
        scratch_types=[
            pltpu.VMEM((BPW * T,), jnp.int32),
            pltpu.VMEM((V * D,), jnp.float32),
            pltpu.VMEM((T * D,), jnp.float32),
            pltpu.VMEM((T * V * D,), jnp.float32),
            pltpu.VMEM((2, T, DT, 8, BT), jnp.float32),
            pltpu.SemaphoreType.DMA,
        ],
        compiler_params=pltpu.CompilerParams(
            use_tc_tiling_on_sc=False, needs_layout_passes=False
        ),
    )
    out5 = sc(x_flat, wt_flat, pos_flat)
    return jnp.transpose(out5, (2, 4, 0, 1, 3)).reshape(B, T, D)


# speedup vs baseline: 14.5652x; 1.3275x over previous
import functools

import jax
import jax.numpy as jnp
from jax import lax
from jax.experimental import pallas as pl
from jax.experimental.pallas import tpu as pltpu
from jax.experimental.pallas import tpu_sc as plsc

B, T, V, D = 16384, 12, 28, 24
LANES = 16

NUM_CORES = 2
NUM_SUBCORES = 16
NW = NUM_CORES * NUM_SUBCORES
BPW = B // NW
BT = 128
UNITS = BPW // BT
DT = D // 8
VP = D + 1
TP = V * VP


def _sc_body(x_hbm, word_hbm, pos_hbm, out_hbm, xch_v, wt_v, pos_v, ft_v, blk_v, sem):
    wid = lax.axis_index("s") * NUM_CORES + lax.axis_index("c")
    b0 = wid * BPW

    pltpu.sync_copy(word_hbm, wt_v.at[pl.ds(0, V * D)])
    pltpu.sync_copy(pos_hbm, pos_v.at[pl.ds(0, T * D)])
    pltpu.sync_copy(x_hbm.at[pl.ds(b0 * T, BPW * T)], xch_v)

    lane = lax.iota(jnp.int32, LANES)
    lane12 = lane * T

    def ft_t(t, _):
        def ft_grp(m, _):
            jr = m * LANES + lane
            q = lax.shift_right_logical(jr * 10486, 18)
            wv = plsc.load_gather(wt_v, [jr - q])
            pvv = plsc.load_gather(pos_v, [jr - q * VP + t * D])
            ft_v[pl.ds(t * TP + m * LANES, LANES)] = wv + pvv
            return 0

        lax.fori_loop(0, (TP + LANES - 1) // LANES, ft_grp, 0)
        return 0

    lax.fori_loop(0, T, ft_t, 0)

    copies = []
    for u in range(UNITS):
        bh = wid * UNITS + u
        xbase = u * BT * T
        buf = u % 2
        if u >= 2:
            copies[u - 2].wait()

        def t_loop(t, _):
            for g in range(BT // LANES):
                xv = plsc.load_gather(xch_v, [lane12 + (xbase + g * (LANES * T) + t)])
                xvt = xv * VP + t * TP
                for d in range(D):
                    val = plsc.load_gather(ft_v, [xvt + d])
                    blk_v[buf, t, d // 8, d % 8, pl.ds(g * LANES, LANES)] = val
            return 0

        lax.fori_loop(0, T, t_loop, 0)
        copies.append(pltpu.async_copy(blk_v.at[buf], out_hbm.at[:, :, bh], sem))
    for c in copies[-2:]:
        c.wait()


@jax.jit
def kernel(X, word_table, pos_table):
    x_flat = X.reshape(B * T)
    wt_flat = word_table.reshape(V * D)
    pos_flat = pos_table.reshape(T * D)

    mesh = plsc.VectorSubcoreMesh(core_axis_name="c", subcore_axis_name="s")
    sc = pl.kernel(
        _sc_body,
        out_type=jax.ShapeDtypeStruct((T, DT, B // BT, 8, BT), jnp.float32),
        mesh=mesh,
        scratch_types=[
            pltpu.VMEM((BPW * T,), jnp.int32),
            pltpu.VMEM((V * D + 32,), jnp.float32),
            pltpu.VMEM((T * D + 16,), jnp.float32),
            pltpu.VMEM((T * TP + 16,), jnp.float32),
            pltpu.VMEM((2, T, DT, 8, BT), jnp.float32),
            pltpu.SemaphoreType.DMA,
        ],
        compiler_params=pltpu.CompilerParams(
            use_tc_tiling_on_sc=False, needs_layout_passes=False
        ),
    )
    out5 = sc(x_flat, wt_flat, pos_flat)
    return jnp.transpose(out5, (2, 4, 0, 1, 3)).reshape(B, T, D)


# speedup vs baseline: 17.6181x; 1.2096x over previous
import functools

import jax
import jax.numpy as jnp
from jax import lax
from jax.experimental import pallas as pl
from jax.experimental.pallas import tpu as pltpu
from jax.experimental.pallas import tpu_sc as plsc

B, T, V, D = 16384, 12, 28, 24
LANES = 16

NUM_CORES = 2
NUM_SUBCORES = 16
NW = NUM_CORES * NUM_SUBCORES
BPW = B // NW
BT = 128
UNITS = BPW // BT
DT = D // 8
VP = D + 1
TP = V * VP


def _sc_body(x_hbm, word_hbm, pos_hbm, out_hbm, xch_v, wt_v, pos_v, ft_v, blk_v, sem):
    wid = lax.axis_index("s") * NUM_CORES + lax.axis_index("c")
    b0 = wid * BPW

    pltpu.sync_copy(word_hbm, wt_v.at[pl.ds(0, V * D)])
    pltpu.sync_copy(pos_hbm, pos_v.at[pl.ds(0, T * D)])
    pltpu.sync_copy(x_hbm.at[:, pl.ds(b0, BPW)], xch_v)

    lane = lax.iota(jnp.int32, LANES)

    def ft_t(t, _):
        def ft_grp(m, _):
            jr = m * LANES + lane
            q = lax.shift_right_logical(jr * 10486, 18)
            wv = plsc.load_gather(wt_v, [jr - q])
            pvv = plsc.load_gather(pos_v, [jr - q * VP + t * D])
            ft_v[pl.ds(t * TP + m * LANES, LANES)] = wv + pvv
            return 0

        lax.fori_loop(0, (TP + LANES - 1) // LANES, ft_grp, 0)
        return 0

    lax.fori_loop(0, T, ft_t, 0)

    copies = []
    for u in range(UNITS):
        bh = wid * UNITS + u
        buf = u % 2
        if u >= 2:
            copies[u - 2].wait()

        def t_loop(t, _):
            for g in range(BT // LANES):
                xv = xch_v[t, pl.ds(u * BT + g * LANES, LANES)]
                xvt = xv * VP + t * TP
                for d in range(D):
                    val = plsc.load_gather(ft_v, [xvt + d])
                    blk_v[buf, t, d // 8, d % 8, pl.ds(g * LANES, LANES)] = val
            return 0

        lax.fori_loop(0, T, t_loop, 0)
        copies.append(pltpu.async_copy(blk_v.at[buf], out_hbm.at[:, :, bh], sem))
    for c in copies[-2:]:
        c.wait()


@jax.jit
def kernel(X, word_table, pos_table):
    x_t = X.T
    wt_flat = word_table.reshape(V * D)
    pos_flat = pos_table.reshape(T * D)

    mesh = plsc.VectorSubcoreMesh(core_axis_name="c", subcore_axis_name="s")
    sc = pl.kernel(
        _sc_body,
        out_type=jax.ShapeDtypeStruct((T, DT, B // BT, 8, BT), jnp.float32),
        mesh=mesh,
        scratch_types=[
            pltpu.VMEM((T, BPW), jnp.int32),
            pltpu.VMEM((V * D + 32,), jnp.float32),
            pltpu.VMEM((T * D + 16,), jnp.float32),
            pltpu.VMEM((T * TP + 16,), jnp.float32),
            pltpu.VMEM((2, T, DT, 8, BT), jnp.float32),
            pltpu.SemaphoreType.DMA,
        ],
        compiler_params=pltpu.CompilerParams(
            use_tc_tiling_on_sc=False, needs_layout_passes=False
        ),
    )
    out5 = sc(x_t, wt_flat, pos_flat)
    return jnp.transpose(out5, (2, 4, 0, 1, 3)).reshape(B, T, D)
